# baseline (speedup 1.0000x reference)
import jax
import jax.numpy as jnp
from jax import lax
from jax.experimental import pallas as pl
from jax.experimental.pallas import tpu as pltpu

B = 8
N = 8192
D = 768
K = 100
C = 20
ITERS = 25
ROWS = 64
LANES = 128


def _body(a0_ref, a1_ref, pt_ref, out_ref, sel_ref, score_ref, idxv_ref,
          idxs_ref, sem):
    pid = pl.program_id(0)

    @pl.when(pid == 0)
    def _topk_gather():
        x0 = a0_ref[...]
        x1 = a1_ref[...]
        m = jnp.maximum(x0, x1)
        e0 = jnp.exp(x0 - m)
        e1 = jnp.exp(x1 - m)
        score_ref[...] = e1 / (e0 + e1)
        lin = (lax.broadcasted_iota(jnp.int32, (B, ROWS, LANES), 1) * LANES
               + lax.broadcasted_iota(jnp.int32, (B, ROWS, LANES), 2))
        lane = lax.broadcasted_iota(jnp.int32, (B, LANES), 1)

        def step(j, acc):
            s = score_ref[...]
            mx = jnp.max(s, axis=(1, 2), keepdims=True)
            cand = jnp.where(s == mx, lin, jnp.int32(1 << 30))
            idx = jnp.min(cand, axis=(1, 2), keepdims=True)
            score_ref[...] = jnp.where(lin == idx, jnp.float32(-1.0), s)
            return jnp.where(lane == j, idx.reshape(B, 1), acc)

        CH = 25
        acc = jnp.zeros((B, LANES), jnp.int32)
        for co in range(K // CH):
            acc = lax.fori_loop(co * CH, (co + 1) * CH, step, acc,
                                unroll=CH)
            idxv_ref[...] = acc
            pltpu.make_async_copy(idxv_ref, idxs_ref, sem).start()
            pltpu.make_async_copy(idxv_ref, idxs_ref, sem).wait()
            for b in range(B):
                def issue(j, _, b=b):
                    idx = idxs_ref[b, j]
                    pltpu.make_async_copy(
                        pt_ref.at[b, pl.ds(idx, 1), :],
                        sel_ref.at[pl.ds(b * K + j, 1), :],
                        sem,
                    ).start()
                    return 0

                lax.fori_loop(co * CH, (co + 1) * CH, issue, 0, unroll=CH)

        def drain(j, _):
            pltpu.make_async_copy(
                pt_ref.at[0, pl.ds(0, 1), :],
                sel_ref.at[pl.ds(0, 1), :],
                sem,
            ).wait()
            return 0

        lax.fori_loop(0, B * K, drain, 0, unroll=20)

    @pl.when(pid == 1)
    def _kmeans():
        sel = sel_ref[...]
        ones = jnp.ones((B * K, 1), jnp.float32)
        selx = jnp.concatenate([sel, ones], axis=1)
        col = lax.broadcasted_iota(jnp.int32, (B * K, B * C), 1)
        rowb = lax.broadcasted_iota(jnp.int32, (B * K, B * C), 0) // K
        valid = (col // C) == rowb
        c0 = jnp.concatenate(
            [sel[b * K:b * K + C] for b in range(B)], axis=0)

        G = 4
        BPG = B // G
        colg = lax.broadcasted_iota(jnp.int32, (BPG * K, BPG * C), 1)
        rowbg = lax.broadcasted_iota(jnp.int32, (BPG * K, BPG * C), 0) // K
        validg = (colg // C) == rowbg

        def labels_of(centers):
            c2 = jnp.sum(centers * centers, axis=1, keepdims=True)
            cext = jnp.concatenate([-2.0 * centers, c2], axis=1)
            labs = []
            for g in range(G):
                d = lax.dot_general(
                    selx[g * BPG * K:(g + 1) * BPG * K],
                    cext[g * BPG * C:(g + 1) * BPG * C],
                    (((1,), (1,)), ((), ())),
                    precision=lax.Precision.HIGHEST,
                    preferred_element_type=jnp.float32)
                d = jnp.where(validg, d, jnp.float32(jnp.inf))
                minv = jnp.min(d, axis=1, keepdims=True)
                labs.append(jnp.min(
                    jnp.where(d == minv, colg, jnp.int32(1 << 30)),
                    axis=1, keepdims=True) + g * BPG * C)
            return jnp.concatenate(labs, axis=0)

        def moments(lab):
            one = (lab == col).astype(jnp.float32)
            se = lax.dot_general(
                one, selx, (((0,), (0,)), ((), ())),
                preferred_element_type=jnp.float32)
            return se[:, :D], se[:, D:D + 1]

        def cond(st):
            i, _, _, changed, _, _ = st
            return (i < ITERS) & changed

        def body(st):
            i, centers, lab_prev, _, _, _ = st
            lab = labels_of(centers)
            sums, counts = moments(lab)
            newc = jnp.where(counts > 0,
                             sums / jnp.maximum(counts, 1.0), centers)
            return (i + 1, newc, lab, jnp.any(lab != lab_prev),
                    sums, counts)

        st0 = (jnp.int32(0), c0,
               jnp.full((B * K, 1), -1, jnp.int32), jnp.bool_(True),
               jnp.zeros((B * C, D), jnp.float32),
               jnp.zeros((B * C, 1), jnp.float32))
        _, centers, _, changed, lsums, lcounts = lax.while_loop(
            cond, body, st0)
        sums, counts = lax.cond(
            changed,
            lambda: moments(labels_of(centers)),
            lambda: (lsums, lcounts))
        cf = sums / jnp.maximum(counts, 1.0)
        norm = jnp.sqrt(jnp.sum(cf * cf, axis=1, keepdims=True))
        out_ref[...] = cf / jnp.maximum(norm, jnp.float32(1e-12))


def kernel(patch_token, anomaly_map, prompt_id):
    del prompt_id
    a0 = anomaly_map[:, :, 0].reshape(B, ROWS, LANES)
    a1 = anomaly_map[:, :, 1].reshape(B, ROWS, LANES)
    out = pl.pallas_call(
        _body,
        grid=(2,),
        in_specs=[
            pl.BlockSpec((B, ROWS, LANES), lambda i: (0, 0, 0)),
            pl.BlockSpec((B, ROWS, LANES), lambda i: (0, 0, 0)),
            pl.BlockSpec(memory_space=pl.ANY),
        ],
        out_specs=pl.BlockSpec((B * C, D), lambda i: (0, 0)),
        out_shape=jax.ShapeDtypeStruct((B * C, D), jnp.float32),
        scratch_shapes=[
            pltpu.VMEM((B * K, D), jnp.float32),
            pltpu.VMEM((B, ROWS, LANES), jnp.float32),
            pltpu.VMEM((B, LANES), jnp.int32),
            pltpu.SMEM((B, LANES), jnp.int32),
            pltpu.SemaphoreType.DMA,
        ],
    )(a0, a1, patch_token)
    return out.reshape(B, C, D)


# speedup vs baseline: 1.0817x; 1.0817x over previous
import jax
import jax.numpy as jnp
from jax import lax
from jax.experimental import pallas as pl
from jax.experimental.pallas import tpu as pltpu

B = 8
N = 8192
D = 768
K = 100
C = 20
ITERS = 25
ROWS = 64
LANES = 128


def _body(a0_ref, a1_ref, pt_ref, out_ref, sel_ref, score_ref, idxv_ref,
          idxs_ref, sem, sem2):
    pid = pl.program_id(0)

    @pl.when(pid == 0)
    def _topk_gather():
        x0 = a0_ref[...]
        x1 = a1_ref[...]
        m = jnp.maximum(x0, x1)
        e0 = jnp.exp(x0 - m)
        e1 = jnp.exp(x1 - m)
        score_ref[...] = e1 / (e0 + e1)
        lin = (lax.broadcasted_iota(jnp.int32, (B, ROWS, LANES), 1) * LANES
               + lax.broadcasted_iota(jnp.int32, (B, ROWS, LANES), 2))
        lane = lax.broadcasted_iota(jnp.int32, (B, LANES), 1)

        def step(j, acc):
            s = score_ref[...]
            mx = jnp.max(s, axis=(1, 2), keepdims=True)
            cand = jnp.where(s == mx, lin, jnp.int32(1 << 30))
            idx = jnp.min(cand, axis=(1, 2), keepdims=True)
            score_ref[...] = jnp.where(lin == idx, jnp.float32(-1.0), s)
            return jnp.where(lane == j, idx.reshape(B, 1), acc)

        CH = 25
        acc = jnp.zeros((B, LANES), jnp.int32)
        for co in range(K // CH):
            acc = lax.fori_loop(co * CH, (co + 1) * CH, step, acc,
                                unroll=CH)
            idxv_ref[...] = acc
            pltpu.make_async_copy(idxv_ref, idxs_ref, sem2).start()
            pltpu.make_async_copy(idxv_ref, idxs_ref, sem2).wait()
            for b in range(B):
                def issue(j, _, b=b):
                    idx = idxs_ref[b, j]
                    pltpu.make_async_copy(
                        pt_ref.at[b, pl.ds(idx, 1), :],
                        sel_ref.at[pl.ds(b * K + j, 1), :],
                        sem,
                    ).start()
                    return 0

                lax.fori_loop(co * CH, (co + 1) * CH, issue, 0, unroll=CH)

        def drain(j, _):
            pltpu.make_async_copy(
                pt_ref.at[0, pl.ds(0, 1), :],
                sel_ref.at[pl.ds(0, 1), :],
                sem,
            ).wait()
            return 0

        lax.fori_loop(0, B * K, drain, 0, unroll=20)

    @pl.when(pid == 1)
    def _kmeans():
        sel = sel_ref[...]
        ones = jnp.ones((B * K, 1), jnp.float32)
        selx = jnp.concatenate([sel, ones], axis=1)
        col = lax.broadcasted_iota(jnp.int32, (B * K, B * C), 1)
        rowb = lax.broadcasted_iota(jnp.int32, (B * K, B * C), 0) // K
        valid = (col // C) == rowb
        c0 = jnp.concatenate(
            [sel[b * K:b * K + C] for b in range(B)], axis=0)

        G = 4
        BPG = B // G
        colg = lax.broadcasted_iota(jnp.int32, (BPG * K, BPG * C), 1)
        rowbg = lax.broadcasted_iota(jnp.int32, (BPG * K, BPG * C), 0) // K
        validg = (colg // C) == rowbg

        def labels_of(centers):
            c2 = jnp.sum(centers * centers, axis=1, keepdims=True)
            cext = jnp.concatenate([-2.0 * centers, c2], axis=1)
            labs = []
            for g in range(G):
                d = lax.dot_general(
                    selx[g * BPG * K:(g + 1) * BPG * K],
                    cext[g * BPG * C:(g + 1) * BPG * C],
                    (((1,), (1,)), ((), ())),
                    precision=lax.Precision.HIGHEST,
                    preferred_element_type=jnp.float32)
                d = jnp.where(validg, d, jnp.float32(jnp.inf))
                minv = jnp.min(d, axis=1, keepdims=True)
                labs.append(jnp.min(
                    jnp.where(d == minv, colg, jnp.int32(1 << 30)),
                    axis=1, keepdims=True) + g * BPG * C)
            return jnp.concatenate(labs, axis=0)

        def moments(lab):
            one = (lab == col).astype(jnp.float32)
            se = lax.dot_general(
                one, selx, (((0,), (0,)), ((), ())),
                preferred_element_type=jnp.float32)
            return se[:, :D], se[:, D:D + 1]

        def cond(st):
            i, _, _, changed, _, _ = st
            return (i < ITERS) & changed

        def body(st):
            i, centers, lab_prev, _, _, _ = st
            lab = labels_of(centers)
            sums, counts = moments(lab)
            newc = jnp.where(counts > 0,
                             sums / jnp.maximum(counts, 1.0), centers)
            return (i + 1, newc, lab, jnp.any(lab != lab_prev),
                    sums, counts)

        st0 = (jnp.int32(0), c0,
               jnp.full((B * K, 1), -1, jnp.int32), jnp.bool_(True),
               jnp.zeros((B * C, D), jnp.float32),
               jnp.zeros((B * C, 1), jnp.float32))
        _, centers, _, changed, lsums, lcounts = lax.while_loop(
            cond, body, st0)
        sums, counts = lax.cond(
            changed,
            lambda: moments(labels_of(centers)),
            lambda: (lsums, lcounts))
        cf = sums / jnp.maximum(counts, 1.0)
        norm = jnp.sqrt(jnp.sum(cf * cf, axis=1, keepdims=True))
        out_ref[...] = cf / jnp.maximum(norm, jnp.float32(1e-12))


def kernel(patch_token, anomaly_map, prompt_id):
    del prompt_id
    a0 = anomaly_map[:, :, 0].reshape(B, ROWS, LANES)
    a1 = anomaly_map[:, :, 1].reshape(B, ROWS, LANES)
    out = pl.pallas_call(
        _body,
        grid=(2,),
        in_specs=[
            pl.BlockSpec((B, ROWS, LANES), lambda i: (0, 0, 0)),
            pl.BlockSpec((B, ROWS, LANES), lambda i: (0, 0, 0)),
            pl.BlockSpec(memory_space=pl.ANY),
        ],
        out_specs=pl.BlockSpec((B * C, D), lambda i: (0, 0)),
        out_shape=jax.ShapeDtypeStruct((B * C, D), jnp.float32),
        scratch_shapes=[
            pltpu.VMEM((B * K, D), jnp.float32),
            pltpu.VMEM((B, ROWS, LANES), jnp.float32),
            pltpu.VMEM((B, LANES), jnp.int32),
            pltpu.SMEM((B, LANES), jnp.int32),
            pltpu.SemaphoreType.DMA,
            pltpu.SemaphoreType.DMA,
        ],
    )(a0, a1, patch_token)
    return out.reshape(B, C, D)
